# Initial kernel scaffold; baseline (speedup 1.0000x reference)
import jax
import jax.numpy as jnp
from jax.experimental import pallas as pl


def kernel(inputs, hidden_state, support, W_ru, b_ru, W_c, b_c):
    raise NotImplementedError("write your pallas kernel here")


# speedup vs baseline: 2.4074x; 2.4074x over previous
import functools

import jax
import jax.numpy as jnp
from jax.experimental import pallas as pl

N_NODES = 325
IN_DIM = 2
UNITS = 64
BATCH = 1024
BT = 16


def _sdot(S, x):
    return jax.lax.dot_general(S, x, (((1,), (0,)), ((), ())),
                               preferred_element_type=jnp.float32)


def _cdot(x, W):
    return jax.lax.dot_general(x, W, (((2,), (0,)), ((), ())),
                               preferred_element_type=jnp.float32)


def _dcgru_kernel(xi_ref, h_ref, S_ref,
                  vx_ru_ref, vh_ru_ref, bru_ref,
                  vx_c_ref, vh_c_ref, bc_ref,
                  out1_ref, out2_ref):
    S = S_ref[...]
    xit = jnp.transpose(xi_ref[...], (1, 0, 2))
    ht = jnp.transpose(h_ref[...], (1, 0, 2))

    xi1 = _sdot(S, xit)
    xi2 = _sdot(S, xi1)

    def gconv(state, vx_ref, vh_ref, b_ref):
        s1 = _sdot(S, state)
        s2 = _sdot(S, s1)
        acc = (_cdot(xit, vx_ref[0]) + _cdot(xi1, vx_ref[1])
               + _cdot(xi2, vx_ref[2])
               + _cdot(state, vh_ref[0]) + _cdot(s1, vh_ref[1])
               + _cdot(s2, vh_ref[2]))
        return acc + b_ref[...]

    ru = jax.nn.sigmoid(gconv(ht, vx_ru_ref, vh_ru_ref, bru_ref))
    r = ru[..., :UNITS]
    u = ru[..., UNITS:]

    c = jnp.tanh(gconv(r * ht, vx_c_ref, vh_c_ref, bc_ref))

    newh = u * ht + (1.0 - u) * c
    out = jnp.transpose(newh, (1, 0, 2))
    out1_ref[...] = out
    out2_ref[...] = out


def _fold_weights(W, out_dim):
    Wm = W.reshape(IN_DIM + UNITS, 3, out_dim)
    V0 = Wm[:, 0, :] - Wm[:, 2, :]
    V1 = Wm[:, 1, :]
    V2 = 2.0 * Wm[:, 2, :]
    V = jnp.stack([V0, V1, V2])
    return V[:, :IN_DIM, :], V[:, IN_DIM:, :]


@jax.jit
def kernel(inputs, hidden_state, support, W_ru, b_ru, W_c, b_c):
    xi = inputs.reshape(BATCH, N_NODES, IN_DIM)
    h = hidden_state[0].reshape(BATCH, N_NODES, UNITS)
    vx_ru, vh_ru = _fold_weights(W_ru, 2 * UNITS)
    vx_c, vh_c = _fold_weights(W_c, UNITS)
    bru = b_ru.reshape(1, 1, 2 * UNITS)
    bc = b_c.reshape(1, 1, UNITS)

    grid = (BATCH // BT,)
    batch_spec = lambda shape: pl.BlockSpec((BT,) + shape,
                                            lambda i: (i, 0, 0))
    full = lambda a: pl.BlockSpec(a.shape, lambda i: (0,) * a.ndim)

    y1, y2 = pl.pallas_call(
        _dcgru_kernel,
        grid=grid,
        in_specs=[
            batch_spec((N_NODES, IN_DIM)),
            batch_spec((N_NODES, UNITS)),
            full(support), full(vx_ru), full(vh_ru), full(bru),
            full(vx_c), full(vh_c), full(bc),
        ],
        out_specs=[batch_spec((N_NODES, UNITS)),
                   batch_spec((N_NODES, UNITS))],
        out_shape=[jax.ShapeDtypeStruct((BATCH, N_NODES, UNITS), jnp.float32),
                   jax.ShapeDtypeStruct((BATCH, N_NODES, UNITS), jnp.float32)],
    )(xi, h, support, vx_ru, vh_ru, bru, vx_c, vh_c, bc)

    output = y1.reshape(BATCH, N_NODES * UNITS)
    hidden = y2.reshape(1, BATCH, N_NODES * UNITS)
    return (output, hidden)


# speedup vs baseline: 2.4818x; 1.0309x over previous
import functools

import jax
import jax.numpy as jnp
from jax.experimental import pallas as pl

N_NODES = 325
IN_DIM = 2
UNITS = 64
BATCH = 1024
BT = 16


def _sdot(S, x):
    return jax.lax.dot_general(S, x.astype(jnp.bfloat16),
                               (((1,), (0,)), ((), ())),
                               preferred_element_type=jnp.float32)


def _cdot(x, W):
    return jax.lax.dot_general(x.astype(jnp.bfloat16), W,
                               (((2,), (0,)), ((), ())),
                               preferred_element_type=jnp.float32)


def _dcgru_kernel(xi_ref, h_ref, S_ref,
                  vx_ru_ref, vh_ru_ref, bru_ref,
                  vx_c_ref, vh_c_ref, bc_ref,
                  out1_ref, out2_ref):
    S = S_ref[...]
    xit = jnp.transpose(xi_ref[...], (1, 0, 2))
    ht = jnp.transpose(h_ref[...], (1, 0, 2))

    xi1 = _sdot(S, xit)
    xi2 = _sdot(S, xi1)

    def gconv(state, vx_ref, vh_ref, b_ref):
        s1 = _sdot(S, state)
        s2 = _sdot(S, s1)
        acc = (_cdot(xit, vx_ref[0]) + _cdot(xi1, vx_ref[1])
               + _cdot(xi2, vx_ref[2])
               + _cdot(state, vh_ref[0]) + _cdot(s1, vh_ref[1])
               + _cdot(s2, vh_ref[2]))
        return acc + b_ref[...]

    ru = jax.nn.sigmoid(gconv(ht, vx_ru_ref, vh_ru_ref, bru_ref))
    r = ru[..., :UNITS]
    u = ru[..., UNITS:]

    c = jnp.tanh(gconv(r * ht, vx_c_ref, vh_c_ref, bc_ref))

    newh = u * ht + (1.0 - u) * c
    out = jnp.transpose(newh, (1, 0, 2))
    out1_ref[...] = out
    out2_ref[...] = out


def _fold_weights(W, out_dim):
    Wm = W.reshape(IN_DIM + UNITS, 3, out_dim)
    V0 = Wm[:, 0, :] - Wm[:, 2, :]
    V1 = Wm[:, 1, :]
    V2 = 2.0 * Wm[:, 2, :]
    V = jnp.stack([V0, V1, V2])
    return V[:, :IN_DIM, :], V[:, IN_DIM:, :]


@jax.jit
def kernel(inputs, hidden_state, support, W_ru, b_ru, W_c, b_c):
    xi = inputs.reshape(BATCH, N_NODES, IN_DIM)
    h = hidden_state[0].reshape(BATCH, N_NODES, UNITS)
    vx_ru, vh_ru = _fold_weights(W_ru, 2 * UNITS)
    vx_c, vh_c = _fold_weights(W_c, UNITS)
    support = support.astype(jnp.bfloat16)
    vx_ru, vh_ru = vx_ru.astype(jnp.bfloat16), vh_ru.astype(jnp.bfloat16)
    vx_c, vh_c = vx_c.astype(jnp.bfloat16), vh_c.astype(jnp.bfloat16)
    bru = b_ru.reshape(1, 1, 2 * UNITS)
    bc = b_c.reshape(1, 1, UNITS)

    grid = (BATCH // BT,)
    batch_spec = lambda shape: pl.BlockSpec((BT,) + shape,
                                            lambda i: (i, 0, 0))
    full = lambda a: pl.BlockSpec(a.shape, lambda i: (0,) * a.ndim)

    y1, y2 = pl.pallas_call(
        _dcgru_kernel,
        grid=grid,
        in_specs=[
            batch_spec((N_NODES, IN_DIM)),
            batch_spec((N_NODES, UNITS)),
            full(support), full(vx_ru), full(vh_ru), full(bru),
            full(vx_c), full(vh_c), full(bc),
        ],
        out_specs=[batch_spec((N_NODES, UNITS)),
                   batch_spec((N_NODES, UNITS))],
        out_shape=[jax.ShapeDtypeStruct((BATCH, N_NODES, UNITS), jnp.float32),
                   jax.ShapeDtypeStruct((BATCH, N_NODES, UNITS), jnp.float32)],
    )(xi, h, support, vx_ru, vh_ru, bru, vx_c, vh_c, bc)

    output = y1.reshape(BATCH, N_NODES * UNITS)
    hidden = y2.reshape(1, BATCH, N_NODES * UNITS)
    return (output, hidden)


# speedup vs baseline: 3.5204x; 1.4185x over previous
import jax
import jax.numpy as jnp
from jax.experimental import pallas as pl

N_NODES = 325
IN_DIM = 2
UNITS = 64
BATCH = 1024
BT = 32
HALF = BT // 2


def _mm(a, b):
    return jax.lax.dot_general(a, b, (((1,), (0,)), ((), ())),
                               preferred_element_type=jnp.float32)


def _rows(a, w):
    return a.reshape(N_NODES, HALF, w).reshape(N_NODES * HALF, w)


def _cols2d(a):
    return a.reshape(N_NODES, HALF, 128).reshape(N_NODES, BT * UNITS)


def _dcgru_kernel(xi_ref, h_ref, S_ref,
                  vh_ru_ref, vx_ru_ref, bru_ref,
                  vh_c_ref, vx_c_ref, bc_ref,
                  out1_ref, out2_ref):
    bf = jnp.bfloat16
    S = S_ref[...]
    h2d = h_ref[...]
    hb = h2d.astype(bf)
    xib = xi_ref[0].astype(bf)

    xi1 = _mm(S, xib)
    xi2 = _mm(S, xi1.astype(bf))
    xis = (xib, xi1.astype(bf), xi2.astype(bf))

    def gconv(st_b, w, vh_ref, vx_ref, b_ref):
        s1 = _mm(S, st_b)
        s2 = _mm(S, s1.astype(bf))
        xc = (_mm(xis[0], vx_ref[0]) + _mm(xis[1], vx_ref[1])
              + _mm(xis[2], vx_ref[2]))
        acc = (_mm(_rows(st_b, 128), vh_ref[0])
               + _mm(_rows(s1.astype(bf), 128), vh_ref[1])
               + _mm(_rows(s2.astype(bf), 128), vh_ref[2])
               + _rows(xc, w))
        return acc + b_ref[...]

    ru = jax.nn.sigmoid(gconv(hb, 256, vh_ru_ref, vx_ru_ref, bru_ref))
    r = ru[:, :128]
    u = ru[:, 128:]

    hr = _rows(h2d, 128)
    st = _cols2d(r * hr).astype(bf)
    c = jnp.tanh(gconv(st, 128, vh_c_ref, vx_c_ref, bc_ref))

    newh = u * hr + (1.0 - u) * c
    out = _cols2d(newh)
    out1_ref[...] = out
    out2_ref[...] = out


def _fold_weights(W, out_dim):
    Wm = W.reshape(IN_DIM + UNITS, 3, out_dim)
    V0 = Wm[:, 0, :] - Wm[:, 2, :]
    V1 = Wm[:, 1, :]
    V2 = 2.0 * Wm[:, 2, :]
    V = jnp.stack([V0, V1, V2])
    return V[:, IN_DIM:, :], V[:, :IN_DIM, :]


def _pack_pair(V, split_gates):
    m, C, O = V.shape
    Z = jnp.zeros_like(V)
    D = jnp.concatenate([jnp.concatenate([V, Z], axis=2),
                         jnp.concatenate([Z, V], axis=2)], axis=1)
    if split_gates:
        D = (D.reshape(m, 2 * C, 2, 2, UNITS)
               .transpose(0, 1, 3, 2, 4)
               .reshape(m, 2 * C, 2 * O))
    return D


@jax.jit
def kernel(inputs, hidden_state, support, W_ru, b_ru, W_c, b_c):
    B, N, U = BATCH, N_NODES, UNITS
    xiT = (inputs.reshape(B, N, IN_DIM).transpose(1, 0, 2)
           .reshape(N, B // BT, BT * IN_DIM).transpose(1, 0, 2))
    hT = hidden_state[0].reshape(B, N, U).transpose(1, 0, 2).reshape(N, B * U)

    bf = jnp.bfloat16
    vh_ru, vx_ru = _fold_weights(W_ru, 2 * U)
    vh_c, vx_c = _fold_weights(W_c, U)
    eye = jnp.eye(HALF, dtype=jnp.float32)
    kron = jax.vmap(lambda v: jnp.kron(eye, v))
    vh_ru = _pack_pair(vh_ru, True).astype(bf)
    vx_ru = kron(_pack_pair(vx_ru, True)).astype(bf)
    vh_c = _pack_pair(vh_c, False).astype(bf)
    vx_c = kron(_pack_pair(vx_c, False)).astype(bf)
    bru = (b_ru.reshape(2, U)[:, None, :]
           .repeat(2, axis=1).reshape(1, 4 * U))
    bc = jnp.tile(b_c, 2).reshape(1, 2 * U)
    Sb = support.astype(jnp.bfloat16)

    grid = (B // BT,)
    col_spec = lambda w: pl.BlockSpec((N, w), lambda i: (0, i))
    full = lambda a: pl.BlockSpec(a.shape, lambda i: (0,) * a.ndim)

    y1, y2 = pl.pallas_call(
        _dcgru_kernel,
        grid=grid,
        in_specs=[
            pl.BlockSpec((1, N, BT * IN_DIM), lambda i: (i, 0, 0)),
            col_spec(BT * U),
            full(Sb), full(vh_ru), full(vx_ru), full(bru),
            full(vh_c), full(vx_c), full(bc),
        ],
        out_specs=[col_spec(BT * U), col_spec(BT * U)],
        out_shape=[jax.ShapeDtypeStruct((N, B * U), jnp.float32),
                   jax.ShapeDtypeStruct((N, B * U), jnp.float32)],
    )(xiT, hT, Sb, vh_ru, vx_ru, bru, vh_c, vx_c, bc)

    output = y1.reshape(N, B, U).transpose(1, 0, 2).reshape(B, N * U)
    hidden = y2.reshape(N, B, U).transpose(1, 0, 2).reshape(1, B, N * U)
    return (output, hidden)


# speedup vs baseline: 5.5603x; 1.5794x over previous
import jax
import jax.numpy as jnp
from jax.experimental import pallas as pl

N_NODES = 325
IN_DIM = 2
UNITS = 64
BATCH = 1024
BT = 32
GRID = BATCH // BT


def _nmix(x, ST):
    b, c, n = x.shape
    y = jax.lax.dot_general(x.reshape(b * c, n), ST, (((1,), (0,)), ((), ())),
                            preferred_element_type=jnp.float32)
    return y.reshape(b, c, n)


def _cmix(w, x):
    return jax.lax.dot_general(w, x, (((1,), (1,)), ((0,), (0,))),
                               preferred_element_type=jnp.float32)


def _dcgru_kernel(xi_ref, h_ref, ST_ref,
                  vh_ru_ref, vx_ru_ref, bru_ref,
                  vh_c_ref, vx_c_ref, bc_ref,
                  out_ref):
    bf = jnp.bfloat16
    ST = ST_ref[...]
    hb = h_ref[...]
    xib = xi_ref[...]

    def bcast(ref, m):
        return jnp.broadcast_to(ref[m][None], (BT,) + ref.shape[1:])

    xi1 = _nmix(xib, ST)
    xi2 = _nmix(xi1.astype(bf), ST)
    xis = (xib, xi1.astype(bf), xi2.astype(bf))

    def gconv(st_b, vh_ref, vx_ref, b_ref):
        s1 = _nmix(st_b, ST)
        s2 = _nmix(s1.astype(bf), ST)
        acc = (_cmix(bcast(vh_ref, 0), st_b)
               + _cmix(bcast(vh_ref, 1), s1.astype(bf))
               + _cmix(bcast(vh_ref, 2), s2.astype(bf))
               + _cmix(bcast(vx_ref, 0), xis[0])
               + _cmix(bcast(vx_ref, 1), xis[1])
               + _cmix(bcast(vx_ref, 2), xis[2]))
        return acc + b_ref[...]

    ru = jax.nn.sigmoid(gconv(hb, vh_ru_ref, vx_ru_ref, bru_ref))
    r = ru[:, :UNITS, :]
    u = ru[:, UNITS:, :]

    st = (r * hb).astype(bf)
    c = jnp.tanh(gconv(st, vh_c_ref, vx_c_ref, bc_ref))

    out_ref[...] = u * hb + (1.0 - u) * c


def _fold_weights(W, out_dim):
    Wm = W.reshape(IN_DIM + UNITS, 3, out_dim)
    V0 = Wm[:, 0, :] - Wm[:, 2, :]
    V1 = Wm[:, 1, :]
    V2 = 2.0 * Wm[:, 2, :]
    V = jnp.stack([V0, V1, V2])
    return V[:, IN_DIM:, :], V[:, :IN_DIM, :]


@jax.jit
def kernel(inputs, hidden_state, support, W_ru, b_ru, W_c, b_c):
    B, N, U, bf = BATCH, N_NODES, UNITS, jnp.bfloat16
    xiT = inputs.reshape(B, N, IN_DIM).transpose(0, 2, 1).astype(bf)
    hT = hidden_state[0].reshape(B, N, U).transpose(0, 2, 1).astype(bf)

    vh_ru, vx_ru = _fold_weights(W_ru, 2 * U)
    vh_c, vx_c = _fold_weights(W_c, U)
    bru = b_ru.reshape(1, 2 * U, 1)
    bc = b_c.reshape(1, U, 1)
    ST = support.T.astype(bf)

    full = lambda a: pl.BlockSpec(a.shape, lambda i: (0,) * a.ndim)
    bspec = lambda c: pl.BlockSpec((BT, c, N), lambda i: (i, 0, 0))

    y = pl.pallas_call(
        _dcgru_kernel,
        grid=(GRID,),
        in_specs=[
            bspec(IN_DIM), bspec(U),
            full(ST), full(vh_ru.astype(bf)), full(vx_ru.astype(bf)),
            full(bru), full(vh_c.astype(bf)), full(vx_c.astype(bf)),
            full(bc),
        ],
        out_specs=bspec(U),
        out_shape=jax.ShapeDtypeStruct((B, U, N), jnp.float32),
    )(xiT, hT, ST, vh_ru.astype(bf), vx_ru.astype(bf), bru,
      vh_c.astype(bf), vx_c.astype(bf), bc)

    output = y.transpose(0, 2, 1).reshape(B, N * U)
    return (output, output[None])
